# baseline (speedup 1.0000x reference)
import functools

import jax
import jax.numpy as jnp
from jax import lax
from jax.experimental import pallas as pl
from jax.experimental.pallas import tpu as pltpu

_NUM_USERS = 1000000
_EMBED_DIM = 64
_BATCH = 16384
_CHUNK = 2048
_K = 8


def _tc_body(idx_hbm, table_hbm, out_vmem, idx_smem, sem_idx, sems):
    nb = _BATCH // _CHUNK

    for b in range(nb):
        pltpu.async_copy(idx_hbm.at[pl.ds(b * _CHUNK, _CHUNK)], idx_smem,
                         sem_idx).wait()

        def fire(i, carry):
            r = idx_smem[i]
            slot = lax.rem(i, _K)
            gi = b * _CHUNK + i

            @pl.when(i >= _K)
            def _():
                pltpu.make_async_copy(table_hbm.at[0],
                                      out_vmem.at[0],
                                      sems.at[slot]).wait()

            pltpu.make_async_copy(table_hbm.at[r], out_vmem.at[gi],
                                  sems.at[slot]).start()
            return carry

        lax.fori_loop(0, _CHUNK, fire, jnp.int32(0))

        def drain(i, carry):
            pltpu.make_async_copy(table_hbm.at[0], out_vmem.at[0],
                                  sems.at[lax.rem(i, _K)]).wait()
            return carry

        lax.fori_loop(0, _K, drain, jnp.int32(0))


@functools.partial(
    pl.pallas_call,
    in_specs=[
        pl.BlockSpec(memory_space=pltpu.HBM),
        pl.BlockSpec(memory_space=pltpu.HBM),
    ],
    out_specs=pl.BlockSpec(memory_space=pltpu.VMEM),
    out_shape=jax.ShapeDtypeStruct((_BATCH, _EMBED_DIM), jnp.float32),
    scratch_shapes=[
        pltpu.SMEM((_CHUNK,), jnp.int32),
        pltpu.SemaphoreType.DMA,
        pltpu.SemaphoreType.DMA((_K,)),
    ],
)
def _tc_gather(idx_hbm, table_hbm, out_vmem, idx_smem, sem_idx, sems):
    _tc_body(idx_hbm, table_hbm, out_vmem, idx_smem, sem_idx, sems)


def kernel(user_indices, embedding_table):
    return _tc_gather(user_indices.astype(jnp.int32), embedding_table)


# speedup vs baseline: 1.0031x; 1.0031x over previous
import functools

import jax
import jax.numpy as jnp
from jax import lax
from jax.experimental import pallas as pl
from jax.experimental.pallas import tpu as pltpu

_NUM_USERS = 1000000
_EMBED_DIM = 64
_BATCH = 16384
_K = 8


def _tc_body(idx_hbm, table_hbm, out_vmem, idx_smem, sem_idx, sems):
    pltpu.async_copy(idx_hbm.at[pl.ds(0, _BATCH)], idx_smem, sem_idx).wait()

    for slot in range(_K):
        pltpu.make_async_copy(table_hbm.at[idx_smem[slot]],
                              out_vmem.at[slot], sems.at[slot]).start()

    def group(g, carry):
        for slot in range(_K):
            pltpu.make_async_copy(table_hbm.at[0], out_vmem.at[0],
                                  sems.at[slot]).wait()
            i = g * _K + slot + _K
            pltpu.make_async_copy(table_hbm.at[idx_smem[i]],
                                  out_vmem.at[i], sems.at[slot]).start()
        return carry

    lax.fori_loop(0, _BATCH // _K - 1, group, jnp.int32(0))

    for slot in range(_K):
        pltpu.make_async_copy(table_hbm.at[0], out_vmem.at[0],
                              sems.at[slot]).wait()


@functools.partial(
    pl.pallas_call,
    in_specs=[
        pl.BlockSpec(memory_space=pltpu.HBM),
        pl.BlockSpec(memory_space=pltpu.HBM),
    ],
    out_specs=pl.BlockSpec(memory_space=pltpu.VMEM),
    out_shape=jax.ShapeDtypeStruct((_BATCH, _EMBED_DIM), jnp.float32),
    scratch_shapes=[
        pltpu.SMEM((_BATCH,), jnp.int32),
        pltpu.SemaphoreType.DMA,
        pltpu.SemaphoreType.DMA((_K,)),
    ],
)
def _tc_gather(idx_hbm, table_hbm, out_vmem, idx_smem, sem_idx, sems):
    _tc_body(idx_hbm, table_hbm, out_vmem, idx_smem, sem_idx, sems)


def kernel(user_indices, embedding_table):
    return _tc_gather(user_indices.astype(jnp.int32), embedding_table)


# speedup vs baseline: 2.7180x; 2.7097x over previous
import functools

import jax
import jax.numpy as jnp
from jax import lax
from jax.experimental import pallas as pl
from jax.experimental.pallas import tpu as pltpu
from jax.experimental.pallas import tpu_sc as plsc

_NUM_USERS = 1000000
_EMBED_DIM = 64
_BATCH = 16384

_SC_ROWS = 13312
_TC_ROWS = _BATCH - _SC_ROWS

_NC = 2
_NS = 16
_NW = _NC * _NS
_B_PER_W = _SC_ROWS // _NW
_L = 16
_NG = _B_PER_W // _L
_K = 8

_mesh = plsc.VectorSubcoreMesh(core_axis_name="c", subcore_axis_name="s")


@functools.partial(
    pl.kernel,
    mesh=_mesh,
    out_type=jax.ShapeDtypeStruct((_SC_ROWS, _EMBED_DIM), jnp.float32),
    scratch_types=[
        pltpu.VMEM((_B_PER_W,), jnp.int32),
        pltpu.VMEM((_B_PER_W, _EMBED_DIM), jnp.float32),
    ] + [pltpu.SemaphoreType.DMA] * 8,
)
def _sc_gather(idx_hbm, table_hbm, out_hbm, idx_v, out_v, *sems):
    wid = lax.axis_index("s") * _NC + lax.axis_index("c")
    base = wid * _B_PER_W

    pltpu.sync_copy(idx_hbm.at[pl.ds(base, _B_PER_W)], idx_v)

    for g in range(_NG):
        rvec = idx_v[pl.ds(g * _L, _L)]
        for l in range(_L):
            pltpu.async_copy(table_hbm.at[rvec[l]], out_v.at[g * _L + l],
                             sems[(g * _L + l) % 8])

    def drain(i, carry):
        for s in range(8):
            pltpu.make_async_copy(table_hbm.at[0], out_v.at[0],
                                  sems[s]).wait()
        return carry

    lax.fori_loop(0, _B_PER_W // 8, drain, jnp.int32(0))

    pltpu.sync_copy(out_v, out_hbm.at[pl.ds(base, _B_PER_W)])


@functools.partial(
    pl.pallas_call,
    in_specs=[
        pl.BlockSpec(memory_space=pltpu.HBM),
        pl.BlockSpec(memory_space=pltpu.HBM),
    ],
    out_specs=pl.BlockSpec(memory_space=pltpu.VMEM),
    out_shape=jax.ShapeDtypeStruct((_TC_ROWS, _EMBED_DIM), jnp.float32),
    scratch_shapes=[
        pltpu.SMEM((_TC_ROWS,), jnp.int32),
        pltpu.SemaphoreType.DMA,
        pltpu.SemaphoreType.DMA((_K,)),
    ],
)
def _tc_gather(idx_hbm, table_hbm, out_vmem, idx_smem, sem_idx, sems):
    pltpu.async_copy(idx_hbm.at[pl.ds(0, _TC_ROWS)], idx_smem,
                     sem_idx).wait()

    for slot in range(_K):
        pltpu.make_async_copy(table_hbm.at[idx_smem[slot]],
                              out_vmem.at[slot], sems.at[slot]).start()

    def group(g, carry):
        for slot in range(_K):
            pltpu.make_async_copy(table_hbm.at[0], out_vmem.at[0],
                                  sems.at[slot]).wait()
            i = g * _K + slot + _K
            pltpu.make_async_copy(table_hbm.at[idx_smem[i]],
                                  out_vmem.at[i], sems.at[slot]).start()
        return carry

    lax.fori_loop(0, _TC_ROWS // _K - 1, group, jnp.int32(0))

    for slot in range(_K):
        pltpu.make_async_copy(table_hbm.at[0], out_vmem.at[0],
                              sems.at[slot]).wait()


def kernel(user_indices, embedding_table):
    idx = user_indices.astype(jnp.int32)
    sc_out = _sc_gather(idx[:_SC_ROWS], embedding_table)
    tc_out = _tc_gather(idx[_SC_ROWS:], embedding_table)
    return jnp.concatenate([sc_out, tc_out], axis=0)


# speedup vs baseline: 4.5643x; 1.6793x over previous
import functools

import jax
import jax.numpy as jnp
from jax import lax
from jax.experimental import pallas as pl
from jax.experimental.pallas import tpu as pltpu
from jax.experimental.pallas import tpu_sc as plsc

_NUM_USERS = 1000000
_EMBED_DIM = 64
_BATCH = 16384

_NC = 2
_NS = 16
_NW = _NC * _NS
_B_PER_W = _BATCH // _NW
_L = 16
_NG = _B_PER_W // _L

_mesh = plsc.VectorSubcoreMesh(core_axis_name="c", subcore_axis_name="s")


@functools.partial(
    pl.kernel,
    mesh=_mesh,
    out_type=jax.ShapeDtypeStruct((_BATCH, _EMBED_DIM), jnp.float32),
    scratch_types=[
        pltpu.VMEM((_B_PER_W,), jnp.int32),
        pltpu.VMEM((_B_PER_W, _EMBED_DIM), jnp.float32),
    ] + [pltpu.SemaphoreType.DMA] * 8,
)
def _gather_kernel(idx_hbm, table_hbm, out_hbm, idx_v, out_v, *sems):
    wid = lax.axis_index("s") * _NC + lax.axis_index("c")
    base = wid * _B_PER_W

    pltpu.sync_copy(idx_hbm.at[pl.ds(base, _B_PER_W)], idx_v)

    for g in range(_NG):
        rvec = idx_v[pl.ds(g * _L, _L)]
        for l in range(_L):
            pltpu.async_copy(table_hbm.at[rvec[l]], out_v.at[g * _L + l],
                             sems[(g * _L + l) % 8])

    def drain(i, carry):
        for s in range(8):
            pltpu.make_async_copy(table_hbm.at[0], out_v.at[0],
                                  sems[s]).wait()
        return carry

    lax.fori_loop(0, _B_PER_W // 8, drain, jnp.int32(0))

    pltpu.sync_copy(out_v, out_hbm.at[pl.ds(base, _B_PER_W)])


def kernel(user_indices, embedding_table):
    return _gather_kernel(user_indices.astype(jnp.int32), embedding_table)
